# Initial kernel scaffold; baseline (speedup 1.0000x reference)
import jax
import jax.numpy as jnp
from jax.experimental import pallas as pl


def kernel(initial_t, w, clause_idx):
    raise NotImplementedError("write your pallas kernel here")


# speedup vs baseline: 4.5112x; 4.5112x over previous
import functools

import jax
import jax.numpy as jnp
from jax import lax
from jax.experimental import pallas as pl
from jax.experimental.pallas import tpu as pltpu
from jax.experimental.pallas import tpu_sc as plsc

B = 64
N = 50000
C = 100000
L = 4
CONV = 0.001
INV_C = 1.0 / C

NT = 16
K = 128
CP = 100352
CPT = CP // NT
NCH = CPT // K
FK = 80
NRC = N // FK


def _sc_call(t01, idxg, idxs, w3, zacc, zcnt, ones8):
    mesh = plsc.VectorSubcoreMesh(core_axis_name="c", subcore_axis_name="s")
    f32 = jnp.float32

    @functools.partial(
        pl.kernel,
        out_type=jax.ShapeDtypeStruct((2 * N, 32), f32),
        mesh=mesh,
        compiler_params=pltpu.CompilerParams(
            use_tc_tiling_on_sc=False, needs_layout_passes=False),
        scratch_types=[
            pltpu.VMEM_SHARED((N + 8, 32), jnp.bfloat16),
            pltpu.VMEM_SHARED((N + 8,), f32),
            pltpu.VMEM_SHARED((NT * 8, 32), f32),
            pltpu.VMEM((K, 32), f32),
            pltpu.VMEM((K, 32), f32),
            pltpu.VMEM((K, 32), f32),
            pltpu.VMEM((K, 32), f32),
            pltpu.VMEM((K, 32), jnp.bfloat16),
            pltpu.VMEM((K,), f32),
            pltpu.VMEM((K,), jnp.int32),
            pltpu.VMEM((K,), jnp.int32),
            pltpu.VMEM((1, 32), f32),
            pltpu.VMEM((8, 32), f32),
            pltpu.VMEM((8, 32), f32),
            pltpu.VMEM((FK, 32), jnp.bfloat16),
            pltpu.VMEM((FK + 16,), f32),
            pltpu.VMEM((FK, 32), f32),
            pltpu.VMEM((FK, 32), f32),
        ],
    )
    def body(t01_h, idxg_h, idxs_h, w_h, zacc_h, zcnt_h, ones8_h, out_h,
             acc_sh, cnt_sh, sat_sh, r0, r1, r2, r3, act, ones_v,
             idxg_v, idxs_v, wv, satv, satall8, accv, cntv, tv, outv):
        c = lax.axis_index("c")
        s = lax.axis_index("s")
        z16 = jnp.zeros((16,), f32)

        nrc_mine = jnp.where(s < NRC - (NRC // NT) * NT, NRC // NT + 1, NRC // NT)
        pltpu.sync_copy(zacc_h, accv)
        pltpu.sync_copy(zcnt_h, cntv.at[pl.ds(0, FK)])
        NZC = (N + 8) // 40

        def zbody(j, _):
            m = s + j * NT
            pltpu.sync_copy(accv.at[pl.ds(0, 40)], acc_sh.at[pl.ds(m * 40, 40)])
            pltpu.sync_copy(cntv.at[pl.ds(0, 40)], cnt_sh.at[pl.ds(m * 40, 40)])
            return 0

        nz_mine = jnp.where(s < NZC - (NZC // NT) * NT, NZC // NT + 1, NZC // NT)
        lax.fori_loop(0, nz_mine, zbody, 0)

        @pl.when(s == 0)
        def _():
            pltpu.sync_copy(accv.at[pl.ds(0, 8)], acc_sh.at[pl.ds(N, 8)])
            pltpu.sync_copy(cntv.at[pl.ds(0, 8)], cnt_sh.at[pl.ds(N, 8)])

        pltpu.sync_copy(ones8_h, ones_v)
        pltpu.sync_copy(w_h.at[c], wv)
        for r in range(8):
            satv[r, pl.ds(0, 16)] = z16
            satv[r, pl.ds(16, 16)] = z16
        plsc.subcore_barrier()

        rbufs = (r0, r1, r2, r3)

        def chunk_body(j, carry):
            sa0, sa1 = carry
            c0 = s * CPT + j * K
            for l in range(L):
                pltpu.sync_copy(
                    idxg_h.at[pl.ds(c * (L * CP) + l * CP + c0, K)], idxg_v)
                pltpu.sync_copy(t01_h.at[idxg_v], rbufs[l])

            def kbody(k, kc):
                ka0, ka1 = kc
                s0 = (r0[k, pl.ds(0, 16)] + r1[k, pl.ds(0, 16)]
                      + r2[k, pl.ds(0, 16)] + r3[k, pl.ds(0, 16)])
                s1 = (r0[k, pl.ds(16, 16)] + r1[k, pl.ds(16, 16)]
                      + r2[k, pl.ds(16, 16)] + r3[k, pl.ds(16, 16)])
                m0 = jnp.where(s0 < 1.0, 1.0, 0.0)
                m1 = jnp.where(s1 < 1.0, 1.0, 0.0)
                act[k, pl.ds(0, 32)] = plsc.pack(
                    m0, m1, format=plsc.PackFormat.INTERLEAVED)
                return (ka0 + jnp.minimum(s0, 1.0), ka1 + jnp.minimum(s1, 1.0))

            sa0, sa1 = lax.fori_loop(0, K, kbody, (sa0, sa1))
            for l in range(L):
                pltpu.sync_copy(idxs_h.at[pl.ds(l * CP + c0, K)], idxs_v)
                pltpu.sync_copy(act, acc_sh.at[idxs_v], add=True)
                pltpu.sync_copy(ones_v, cnt_sh.at[idxs_v], add=True)
            return sa0, sa1

        sa0, sa1 = lax.fori_loop(0, NCH, chunk_body, (z16, z16))
        satv[0, pl.ds(0, 16)] = sa0
        satv[0, pl.ds(16, 16)] = sa1
        pltpu.sync_copy(satv, sat_sh.at[pl.ds(s * 8, 8)])
        plsc.subcore_barrier()

        def sumb(r, acc2):
            pltpu.sync_copy(sat_sh.at[pl.ds(r * 8, 8)], satall8)
            return (acc2[0] + satall8[0, pl.ds(0, 16)],
                    acc2[1] + satall8[0, pl.ds(16, 16)])

        t0s, t1s = lax.fori_loop(0, NT, sumb, (z16, z16))
        sat0 = t0s * INV_C
        sat1 = t1s * INV_C
        w0 = wv[0, pl.ds(0, 16)]
        w1 = wv[0, pl.ds(16, 16)]
        d0 = w0 - sat0
        d1 = w1 - sat1
        dsC0 = jnp.where(jnp.abs(d0) > CONV, d0, 0.0) * INV_C
        dsC1 = jnp.where(jnp.abs(d1) > CONV, d1, 0.0) * INV_C

        def fin(j, _):
            m = s + j * NT
            rn = m * FK
            def rda(q, __):
                pltpu.sync_copy(acc_sh.at[pl.ds(rn + q * 40, 40)],
                                accv.at[pl.ds(q * 40, 40)])
                return 0

            lax.fori_loop(0, FK // 40, rda, 0)
            pltpu.sync_copy(cnt_sh.at[pl.ds(rn, FK)], cntv.at[pl.ds(0, FK)])
            pltpu.sync_copy(t01_h.at[pl.ds(c * N + rn, FK)], tv)

            def fb(i, __):
                cwin = cntv[pl.ds(i, 16)]
                cv = jnp.full((16,), cwin[0], f32)
                recip = 1.0 / jnp.maximum(cv, 1.0)
                u0, u1 = plsc.unpack(accv[i, pl.ds(0, 32)],
                                     format=plsc.PackFormat.INTERLEAVED)
                o0 = tv[i, pl.ds(0, 16)] + u0 * dsC0 * recip
                o1 = tv[i, pl.ds(16, 16)] + u1 * dsC1 * recip
                outv[i, pl.ds(0, 16)] = jnp.minimum(jnp.maximum(o0, 0.0), 1.0)
                outv[i, pl.ds(16, 16)] = jnp.minimum(jnp.maximum(o1, 0.0), 1.0)
                return 0

            lax.fori_loop(0, FK, fb, 0)
            pltpu.sync_copy(outv, out_h.at[pl.ds(c * N + rn, FK)])
            return 0

        lax.fori_loop(0, nrc_mine, fin, 0)

    return body(t01, idxg, idxs, w3, zacc, zcnt, ones8)


def kernel(initial_t, w, clause_idx):
    f32 = jnp.float32
    t3 = initial_t.reshape(2, 32, N).transpose(0, 2, 1).reshape(2 * N, 32)
    t01 = jnp.concatenate([t3, jnp.zeros((8, 32), f32)], axis=0)

    cit = clause_idx.T.astype(jnp.int32)
    pad_s = jnp.full((L, CP - C), N, jnp.int32)
    pad_g = jnp.full((L, CP - C), 2 * N, jnp.int32)
    idx_s = jnp.concatenate([cit, pad_s], axis=1).reshape(-1)
    idx_g = jnp.stack([
        jnp.concatenate([cit, pad_g], axis=1),
        jnp.concatenate([cit + N, pad_g], axis=1),
    ]).reshape(-1)

    zacc = jnp.zeros((FK, 32), jnp.bfloat16)
    zcnt = jnp.zeros((FK,), f32)
    ones8 = jnp.ones((K,), f32)
    w3 = w.astype(f32).reshape(2, 1, 32)

    out01 = _sc_call(t01, idx_g, idx_s, w3, zacc, zcnt, ones8)
    new_t = out01.reshape(2, N, 32).transpose(0, 2, 1).reshape(B, N)
    return jnp.stack([initial_t, new_t])
